# baseline (speedup 1.0000x reference)
import functools

import jax
import jax.numpy as jnp
from jax import lax
from jax.experimental import pallas as pl
from jax.experimental.pallas import tpu as pltpu
from jax.experimental.pallas import tpu_sc as plsc

_EPS = 1e-05
_MAX_NORM = 1.0 - _EPS

_NC = 2
_NS = 16
_NW = _NC * _NS
_LANES = 16
_CHUNK = 128
_PACK = 4


@functools.lru_cache(maxsize=None)
def _make_sc_pack(v, dim):
    assert dim == 32
    full_blocks = v // 128
    tail_cols = v - full_blocks * 128
    base_blocks = (full_blocks // (4 * _NW)) * 4
    extra_blocks = full_blocks - base_blocks * _NW
    assert extra_blocks <= _NW
    npairs = base_blocks // 2
    assert tail_cols % (2 * _PACK) == 0

    mesh = plsc.VectorSubcoreMesh(core_axis_name="c", subcore_axis_name="s")
    out_t = jax.ShapeDtypeStruct((v // _PACK, _PACK * dim), jnp.float32)
    scratch = (
        [pltpu.VMEM((dim, 128), jnp.float32) for _ in range(4)]
        + [pltpu.VMEM((8, 128), jnp.float32) for _ in range(16)]
        + [pltpu.VMEM((dim, tail_cols), jnp.float32) if tail_cols
           else pltpu.VMEM((dim, _PACK), jnp.float32)]
        + [pltpu.SemaphoreType.DMA] * 4
    )

    @functools.partial(pl.kernel, mesh=mesh, out_type=out_t,
                       scratch_types=scratch,
                       compiler_params=pltpu.CompilerParams(
                           needs_layout_passes=False))
    def pack(wt_hbm, out_hbm, *refs):
        blkA0, blkA1, blkB0, blkB1 = refs[0:4]
        pkb = refs[4:20]
        tail_v = refs[20]
        semInA, semInB, semOutA, semOutB = refs[21:25]
        wid = lax.axis_index("s") * _NC + lax.axis_index("c")
        lanes = lax.iota(jnp.int32, _LANES)
        g0 = wid * base_blocks

        def fire_in(g, b0, b1, sem):
            pltpu.async_copy(wt_hbm.at[:, pl.ds(g * 128, 128)], b0, sem)
            pltpu.async_copy(wt_hbm.at[:, pl.ds((g + 1) * 128, 128)], b1, sem)

        def drain_in(b0, b1, sem):
            pltpu.make_async_copy(
                wt_hbm.at[:, pl.ds(0, 128)], b0, sem).wait()
            pltpu.make_async_copy(
                wt_hbm.at[:, pl.ds(0, 128)], b1, sem).wait()

        def permute_one(blk, pset, bi):
            @plsc.parallel_loop(0, dim // _LANES)
            def ddbody(dd):
                dvec = dd * _LANES + lanes
                for w in range(8):
                    q = 4 * bi + (w >> 1)
                    for r in range(_LANES):
                        xv = w * _LANES + ((lanes + r) & (_LANES - 1))
                        row = (xv >> 2) - 4 * (w & ~1)
                        plsc.store_scatter(
                            pset[q], [row, (xv & 3) * dim + dvec],
                            plsc.load_gather(blk, [dvec, xv]))

        def permute_pair(b0, b1, pset):
            permute_one(b0, pset, 0)
            permute_one(b1, pset, 1)

        def fire_out(grow, pset, sem):
            for q in range(8):
                pltpu.async_copy(
                    pset[q], out_hbm.at[pl.ds(grow + 8 * q, 8)], sem)

        def drain_out(pset, sem):
            for q in range(8):
                pltpu.make_async_copy(
                    pset[q], out_hbm.at[pl.ds(0, 8)], sem).wait()

        fire_in(g0, blkA0, blkA1, semInA)

        def body(u, carry):
            gA = g0 + 4 * u
            fire_in(gA + 2, blkB0, blkB1, semInB)

            @pl.when(u > 0)
            def _():
                drain_out(pkb[0:8], semOutA)
            drain_in(blkA0, blkA1, semInA)
            permute_pair(blkA0, blkA1, pkb[0:8])
            fire_out(gA * 32, pkb[0:8], semOutA)

            gnext = jnp.minimum(gA + 4, g0 + base_blocks - 4)
            fire_in(gnext, blkA0, blkA1, semInA)

            @pl.when(u > 0)
            def _():
                drain_out(pkb[8:16], semOutB)
            drain_in(blkB0, blkB1, semInB)
            permute_pair(blkB0, blkB1, pkb[8:16])
            fire_out((gA + 2) * 32, pkb[8:16], semOutB)
            return carry

        lax.fori_loop(0, npairs // 2, body, 0)
        drain_in(blkA0, blkA1, semInA)
        drain_out(pkb[0:8], semOutA)
        drain_out(pkb[8:16], semOutB)

        if extra_blocks:
            @pl.when(wid >= _NW - extra_blocks)
            def _():
                g = base_blocks * _NW + (wid - (_NW - extra_blocks))
                pltpu.sync_copy(wt_hbm.at[:, pl.ds(g * 128, 128)], blkA0)
                permute_one(blkA0, pkb[0:8], 0)
                for q in range(4):
                    pltpu.sync_copy(
                        pkb[q], out_hbm.at[pl.ds(g * 32 + 8 * q, 8)])

        if tail_cols:
            @pl.when(wid == _NW - 1)
            def _():
                pltpu.sync_copy(
                    wt_hbm.at[:, pl.ds(full_blocks * 128, tail_cols)], tail_v)

                def tddbody(dd, c):
                    dvec = dd * _LANES + lanes
                    for w in range(tail_cols // _LANES):
                        q = w >> 1
                        for r in range(_LANES):
                            xv = w * _LANES + ((lanes + r) & (_LANES - 1))
                            row = (xv >> 2) - 4 * (w & ~1)
                            plsc.store_scatter(
                                pkb[q], [row, (xv & 3) * dim + dvec],
                                plsc.load_gather(tail_v, [dvec, xv]))
                    return c
                lax.fori_loop(0, dim // _LANES, tddbody, 0)
                nq = tail_cols // (2 * _LANES)
                for q in range(nq):
                    pltpu.sync_copy(
                        pkb[q],
                        out_hbm.at[pl.ds(full_blocks * 32 + 8 * q, 8)])

    return pack


@functools.lru_cache(maxsize=None)
def _make_sc_stats(n, dim):
    assert n % (_NW * _CHUNK) == 0
    assert dim & (dim - 1) == 0
    npw = n // _NW
    nchunk = npw // _CHUNK
    assert nchunk % 2 == 0
    ngroup = _CHUNK // _LANES
    line = _PACK * dim

    mesh = plsc.VectorSubcoreMesh(core_axis_name="c", subcore_axis_name="s")
    out_t = [jax.ShapeDtypeStruct((n,), jnp.float32)] * 3
    scratch = [
        pltpu.VMEM((npw,), jnp.int32),
        pltpu.VMEM((npw,), jnp.int32),
        pltpu.VMEM((npw,), jnp.int32),
        pltpu.VMEM((npw,), jnp.int32),
        pltpu.VMEM((_CHUNK, line), jnp.float32),
        pltpu.VMEM((_CHUNK, line), jnp.float32),
        pltpu.VMEM((_CHUNK, line), jnp.float32),
        pltpu.VMEM((_CHUNK, line), jnp.float32),
        pltpu.VMEM((npw,), jnp.float32),
        pltpu.VMEM((npw,), jnp.float32),
        pltpu.VMEM((npw,), jnp.float32),
        pltpu.SemaphoreType.DMA,
        pltpu.SemaphoreType.DMA,
    ]

    @functools.partial(pl.kernel, mesh=mesh, out_type=out_t,
                       scratch_types=scratch,
                       compiler_params=pltpu.CompilerParams(
                           needs_layout_passes=False))
    def sc(wq_hbm, x_hbm, y_hbm, jx_hbm, jy_hbm,
           su_hbm, sv_hbm, suv_hbm,
           idx_x, idx_y, jx_v, jy_v,
           rx0, ry0, rx1, ry1, su_v, sv_v, suv_v, sem0, sem1):
        wid = lax.axis_index("s") * _NC + lax.axis_index("c")
        lanes = lax.iota(jnp.int32, _LANES)
        woff = wid * npw

        pltpu.sync_copy(x_hbm.at[pl.ds(woff, npw)], idx_x)
        pltpu.sync_copy(y_hbm.at[pl.ds(woff, npw)], idx_y)
        pltpu.sync_copy(jx_hbm.at[pl.ds(woff, npw)], jx_v)
        pltpu.sync_copy(jy_hbm.at[pl.ds(woff, npw)], jy_v)

        def fire(c, rx, ry, sem):
            pltpu.async_copy(
                wq_hbm.at[jx_v.at[pl.ds(c * _CHUNK, _CHUNK)]], rx, sem)
            pltpu.async_copy(
                wq_hbm.at[jy_v.at[pl.ds(c * _CHUNK, _CHUNK)]], ry, sem)

        def drain(rx, ry, sem):
            pltpu.make_async_copy(
                wq_hbm.at[jx_v.at[pl.ds(0, _CHUNK)]], rx, sem).wait()
            pltpu.make_async_copy(
                wq_hbm.at[jy_v.at[pl.ds(0, _CHUNK)]], ry, sem).wait()

        def compute(c, rx, ry):
            def group_body(g, gc):
                base = g * _LANES
                pos = c * _CHUNK + base
                p = base + lanes
                offx = (idx_x[pl.ds(pos, _LANES)] & (_PACK - 1)) * dim
                offy = (idx_y[pl.ds(pos, _LANES)] & (_PACK - 1)) * dim
                su = jnp.zeros((_LANES,), jnp.float32)
                sv = jnp.zeros((_LANES,), jnp.float32)
                suv = jnp.zeros((_LANES,), jnp.float32)
                for d in range(dim):
                    col = (lanes + d) & (dim - 1)
                    vx = plsc.load_gather(rx, [p, offx + col])
                    vy = plsc.load_gather(ry, [p, offy + col])
                    su = su + vx * vx
                    sv = sv + vy * vy
                    df = vx - vy
                    suv = suv + df * df
                su_v[pl.ds(pos, _LANES)] = su
                sv_v[pl.ds(pos, _LANES)] = sv
                suv_v[pl.ds(pos, _LANES)] = suv
                return gc
            lax.fori_loop(0, ngroup, group_body, 0)

        fire(0, rx0, ry0, sem0)

        def pair_body(t, carry):
            c = 2 * t
            fire(c + 1, rx1, ry1, sem1)
            drain(rx0, ry0, sem0)
            compute(c, rx0, ry0)
            fire(jnp.minimum(c + 2, nchunk - 1), rx0, ry0, sem0)
            drain(rx1, ry1, sem1)
            compute(c + 1, rx1, ry1)
            return carry

        lax.fori_loop(0, nchunk // 2, pair_body, 0)
        drain(rx0, ry0, sem0)

        pltpu.sync_copy(su_v, su_hbm.at[pl.ds(woff, npw)])
        pltpu.sync_copy(sv_v, sv_hbm.at[pl.ds(woff, npw)])
        pltpu.sync_copy(suv_v, suv_hbm.at[pl.ds(woff, npw)])

    return sc


def _tc_dist(su_ref, sv_ref, suv_ref, o_ref):
    su = su_ref[...]
    sv = sv_ref[...]
    suv = suv_ref[...]
    cu = jnp.minimum(1.0, _MAX_NORM / jnp.maximum(jnp.sqrt(su), 1e-12))
    cv = jnp.minimum(1.0, _MAX_NORM / jnp.maximum(jnp.sqrt(sv), 1e-12))
    dt = 0.5 * (su + sv - suv)
    clamped = jnp.logical_or(cu < 1.0, cv < 1.0)
    suv_eff = jnp.where(
        clamped,
        jnp.maximum(cu * cu * su + cv * cv * sv - 2.0 * cu * cv * dt, 0.0),
        suv)
    norm_u = cu * jnp.sqrt(su)
    norm_v = cv * jnp.sqrt(sv)
    norm_uv = jnp.sqrt(suv_eff)
    d = 1 + 2 * norm_uv ** 2 / ((1 - norm_u ** 2) * (1 - norm_v ** 2))
    o_ref[...] = jnp.log(d + jnp.sqrt((d + 1.0) * (d - 1.0)))


def kernel(x, y, weight):
    b, l = x.shape
    n = b * l
    v, dim = weight.shape
    xf = x.reshape(n).astype(jnp.int32)
    yf = y.reshape(n).astype(jnp.int32)
    w = weight.astype(jnp.float32)
    wq = _make_sc_pack(v, dim)(w.T)
    jx = xf // _PACK
    jy = yf // _PACK
    su, sv, suv = _make_sc_stats(n, dim)(wq, xf, yf, jx, jy)
    shape2 = (n // 128, 128)
    dist = pl.pallas_call(
        _tc_dist,
        out_shape=jax.ShapeDtypeStruct(shape2, jnp.float32),
    )(su.reshape(shape2), sv.reshape(shape2), suv.reshape(shape2))
    return dist.reshape(b, l)


# speedup vs baseline: 1.0342x; 1.0342x over previous
import functools

import jax
import jax.numpy as jnp
from jax import lax
from jax.experimental import pallas as pl
from jax.experimental.pallas import tpu as pltpu
from jax.experimental.pallas import tpu_sc as plsc

_EPS = 1e-05
_MAX_NORM = 1.0 - _EPS

_NC = 2
_NS = 16
_NW = _NC * _NS
_LANES = 16
_CHUNK = 128
_PACK = 4


@functools.lru_cache(maxsize=None)
def _make_sc_pack(v, dim):
    assert dim == 32
    full_blocks = v // 128
    tail_cols = v - full_blocks * 128
    base_blocks = (full_blocks // (4 * _NW)) * 4
    extra_blocks = full_blocks - base_blocks * _NW
    assert extra_blocks <= _NW
    npairs = base_blocks // 2
    assert tail_cols % (2 * _PACK) == 0

    mesh = plsc.VectorSubcoreMesh(core_axis_name="c", subcore_axis_name="s")
    out_t = jax.ShapeDtypeStruct((v // _PACK, _PACK * dim), jnp.float32)
    scratch = (
        [pltpu.VMEM((dim, 256), jnp.float32) for _ in range(2)]
        + [pltpu.VMEM((dim, 128), jnp.float32)]
        + [pltpu.VMEM((8, 128), jnp.float32) for _ in range(16)]
        + [pltpu.VMEM((dim, tail_cols), jnp.float32) if tail_cols
           else pltpu.VMEM((dim, _PACK), jnp.float32)]
        + [pltpu.SemaphoreType.DMA] * 4
    )

    @functools.partial(pl.kernel, mesh=mesh, out_type=out_t,
                       scratch_types=scratch,
                       compiler_params=pltpu.CompilerParams(
                           needs_layout_passes=False))
    def pack(wt_hbm, out_hbm, *refs):
        blkA, blkB, blkE = refs[0:3]
        pkb = refs[3:19]
        tail_v = refs[19]
        semInA, semInB, semOutA, semOutB = refs[20:24]
        wid = lax.axis_index("s") * _NC + lax.axis_index("c")
        lanes = lax.iota(jnp.int32, _LANES)
        g0 = wid * base_blocks

        def fire_in(g, big, sem):
            pltpu.async_copy(wt_hbm.at[:, pl.ds(g * 128, 256)], big, sem)

        def drain_in(big, sem):
            pltpu.make_async_copy(
                wt_hbm.at[:, pl.ds(0, 256)], big, sem).wait()

        def permute_one(blk, pset, bi, xoff):
            def ddbody(dd, c):
                dvec = dd * _LANES + lanes
                for w in range(8):
                    q = 4 * bi + (w >> 1)
                    for r in range(_LANES):
                        xv = w * _LANES + ((lanes + r) & (_LANES - 1))
                        row = (xv >> 2) - 4 * (w & ~1)
                        plsc.store_scatter(
                            pset[q], [row, (xv & 3) * dim + dvec],
                            plsc.load_gather(blk, [dvec, xoff + xv]))
                return c
            lax.fori_loop(0, dim // _LANES, ddbody, 0)

        def permute_pair(big, pset):
            permute_one(big, pset, 0, 0)
            permute_one(big, pset, 1, 128)

        def fire_out(grow, pset, sem):
            for q in range(8):
                pltpu.async_copy(
                    pset[q], out_hbm.at[pl.ds(grow + 8 * q, 8)], sem)

        def drain_out(pset, sem):
            for q in range(8):
                pltpu.make_async_copy(
                    pset[q], out_hbm.at[pl.ds(0, 8)], sem).wait()

        fire_in(g0, blkA, semInA)

        def body(u, carry):
            gA = g0 + 4 * u
            fire_in(gA + 2, blkB, semInB)

            @pl.when(u > 0)
            def _():
                drain_out(pkb[0:8], semOutA)
            drain_in(blkA, semInA)
            permute_pair(blkA, pkb[0:8])
            fire_out(gA * 32, pkb[0:8], semOutA)

            gnext = jnp.minimum(gA + 4, g0 + base_blocks - 4)
            fire_in(gnext, blkA, semInA)

            @pl.when(u > 0)
            def _():
                drain_out(pkb[8:16], semOutB)
            drain_in(blkB, semInB)
            permute_pair(blkB, pkb[8:16])
            fire_out((gA + 2) * 32, pkb[8:16], semOutB)
            return carry

        lax.fori_loop(0, npairs // 2, body, 0)
        drain_in(blkA, semInA)
        drain_out(pkb[0:8], semOutA)
        drain_out(pkb[8:16], semOutB)

        if extra_blocks:
            @pl.when(wid >= _NW - extra_blocks)
            def _():
                g = base_blocks * _NW + (wid - (_NW - extra_blocks))
                pltpu.sync_copy(wt_hbm.at[:, pl.ds(g * 128, 128)], blkE)
                permute_one(blkE, pkb[0:8], 0, 0)
                for q in range(4):
                    pltpu.sync_copy(
                        pkb[q], out_hbm.at[pl.ds(g * 32 + 8 * q, 8)])

        if tail_cols:
            @pl.when(wid == _NW - 1)
            def _():
                pltpu.sync_copy(
                    wt_hbm.at[:, pl.ds(full_blocks * 128, tail_cols)], tail_v)

                def tddbody(dd, c):
                    dvec = dd * _LANES + lanes
                    for w in range(tail_cols // _LANES):
                        q = w >> 1
                        for r in range(_LANES):
                            xv = w * _LANES + ((lanes + r) & (_LANES - 1))
                            row = (xv >> 2) - 4 * (w & ~1)
                            plsc.store_scatter(
                                pkb[q], [row, (xv & 3) * dim + dvec],
                                plsc.load_gather(tail_v, [dvec, xv]))
                    return c
                lax.fori_loop(0, dim // _LANES, tddbody, 0)
                nq = tail_cols // (2 * _LANES)
                for q in range(nq):
                    pltpu.sync_copy(
                        pkb[q],
                        out_hbm.at[pl.ds(full_blocks * 32 + 8 * q, 8)])

    return pack


@functools.lru_cache(maxsize=None)
def _make_sc_stats(n, dim):
    assert n % (_NW * _CHUNK) == 0
    assert dim & (dim - 1) == 0
    npw = n // _NW
    nchunk = npw // _CHUNK
    assert nchunk % 2 == 0
    ngroup = _CHUNK // _LANES
    line = _PACK * dim

    mesh = plsc.VectorSubcoreMesh(core_axis_name="c", subcore_axis_name="s")
    out_t = [jax.ShapeDtypeStruct((n,), jnp.float32)] * 3
    scratch = [
        pltpu.VMEM((npw,), jnp.int32),
        pltpu.VMEM((npw,), jnp.int32),
        pltpu.VMEM((npw,), jnp.int32),
        pltpu.VMEM((npw,), jnp.int32),
        pltpu.VMEM((_CHUNK, line), jnp.float32),
        pltpu.VMEM((_CHUNK, line), jnp.float32),
        pltpu.VMEM((_CHUNK, line), jnp.float32),
        pltpu.VMEM((_CHUNK, line), jnp.float32),
        pltpu.VMEM((npw,), jnp.float32),
        pltpu.VMEM((npw,), jnp.float32),
        pltpu.VMEM((npw,), jnp.float32),
        pltpu.SemaphoreType.DMA,
        pltpu.SemaphoreType.DMA,
    ]

    @functools.partial(pl.kernel, mesh=mesh, out_type=out_t,
                       scratch_types=scratch,
                       compiler_params=pltpu.CompilerParams(
                           needs_layout_passes=False))
    def sc(wq_hbm, x_hbm, y_hbm, jx_hbm, jy_hbm,
           su_hbm, sv_hbm, suv_hbm,
           idx_x, idx_y, jx_v, jy_v,
           rx0, ry0, rx1, ry1, su_v, sv_v, suv_v, sem0, sem1):
        wid = lax.axis_index("s") * _NC + lax.axis_index("c")
        lanes = lax.iota(jnp.int32, _LANES)
        woff = wid * npw

        pltpu.sync_copy(x_hbm.at[pl.ds(woff, npw)], idx_x)
        pltpu.sync_copy(y_hbm.at[pl.ds(woff, npw)], idx_y)
        pltpu.sync_copy(jx_hbm.at[pl.ds(woff, npw)], jx_v)
        pltpu.sync_copy(jy_hbm.at[pl.ds(woff, npw)], jy_v)

        def fire(c, rx, ry, sem):
            pltpu.async_copy(
                wq_hbm.at[jx_v.at[pl.ds(c * _CHUNK, _CHUNK)]], rx, sem)
            pltpu.async_copy(
                wq_hbm.at[jy_v.at[pl.ds(c * _CHUNK, _CHUNK)]], ry, sem)

        def drain(rx, ry, sem):
            pltpu.make_async_copy(
                wq_hbm.at[jx_v.at[pl.ds(0, _CHUNK)]], rx, sem).wait()
            pltpu.make_async_copy(
                wq_hbm.at[jy_v.at[pl.ds(0, _CHUNK)]], ry, sem).wait()

        def compute(c, rx, ry):
            def group_body(g, gc):
                base = g * _LANES
                pos = c * _CHUNK + base
                p = base + lanes
                offx = (idx_x[pl.ds(pos, _LANES)] & (_PACK - 1)) * dim
                offy = (idx_y[pl.ds(pos, _LANES)] & (_PACK - 1)) * dim
                su = jnp.zeros((_LANES,), jnp.float32)
                sv = jnp.zeros((_LANES,), jnp.float32)
                suv = jnp.zeros((_LANES,), jnp.float32)
                for d in range(dim):
                    col = (lanes + d) & (dim - 1)
                    vx = plsc.load_gather(rx, [p, offx + col])
                    vy = plsc.load_gather(ry, [p, offy + col])
                    su = su + vx * vx
                    sv = sv + vy * vy
                    df = vx - vy
                    suv = suv + df * df
                su_v[pl.ds(pos, _LANES)] = su
                sv_v[pl.ds(pos, _LANES)] = sv
                suv_v[pl.ds(pos, _LANES)] = suv
                return gc
            lax.fori_loop(0, ngroup, group_body, 0)

        fire(0, rx0, ry0, sem0)

        def pair_body(t, carry):
            c = 2 * t
            fire(c + 1, rx1, ry1, sem1)
            drain(rx0, ry0, sem0)
            compute(c, rx0, ry0)
            fire(jnp.minimum(c + 2, nchunk - 1), rx0, ry0, sem0)
            drain(rx1, ry1, sem1)
            compute(c + 1, rx1, ry1)
            return carry

        lax.fori_loop(0, nchunk // 2, pair_body, 0)
        drain(rx0, ry0, sem0)

        pltpu.sync_copy(su_v, su_hbm.at[pl.ds(woff, npw)])
        pltpu.sync_copy(sv_v, sv_hbm.at[pl.ds(woff, npw)])
        pltpu.sync_copy(suv_v, suv_hbm.at[pl.ds(woff, npw)])

    return sc


def _tc_dist(su_ref, sv_ref, suv_ref, o_ref):
    su = su_ref[...]
    sv = sv_ref[...]
    suv = suv_ref[...]
    cu = jnp.minimum(1.0, _MAX_NORM / jnp.maximum(jnp.sqrt(su), 1e-12))
    cv = jnp.minimum(1.0, _MAX_NORM / jnp.maximum(jnp.sqrt(sv), 1e-12))
    dt = 0.5 * (su + sv - suv)
    clamped = jnp.logical_or(cu < 1.0, cv < 1.0)
    suv_eff = jnp.where(
        clamped,
        jnp.maximum(cu * cu * su + cv * cv * sv - 2.0 * cu * cv * dt, 0.0),
        suv)
    norm_u = cu * jnp.sqrt(su)
    norm_v = cv * jnp.sqrt(sv)
    norm_uv = jnp.sqrt(suv_eff)
    d = 1 + 2 * norm_uv ** 2 / ((1 - norm_u ** 2) * (1 - norm_v ** 2))
    o_ref[...] = jnp.log(d + jnp.sqrt((d + 1.0) * (d - 1.0)))


def kernel(x, y, weight):
    b, l = x.shape
    n = b * l
    v, dim = weight.shape
    xf = x.reshape(n).astype(jnp.int32)
    yf = y.reshape(n).astype(jnp.int32)
    w = weight.astype(jnp.float32)
    wq = _make_sc_pack(v, dim)(w.T)
    jx = xf // _PACK
    jy = yf // _PACK
    su, sv, suv = _make_sc_stats(n, dim)(wq, xf, yf, jx, jy)
    shape2 = (n // 128, 128)
    dist = pl.pallas_call(
        _tc_dist,
        out_shape=jax.ShapeDtypeStruct(shape2, jnp.float32),
    )(su.reshape(shape2), sv.reshape(shape2), suv.reshape(shape2))
    return dist.reshape(b, l)
